# baseline (device time: 407806 ns/iter reference)
import jax
import jax.numpy as jnp
from jax import lax
from jax.experimental import pallas as pl
from jax.experimental.pallas import tpu as pltpu

N_DEV = 16


def kernel(x, w_mat):
    m, k = x.shape
    _, n = w_mat.shape
    ch = m // N_DEV

    def body(x_ref, w_ref, out_ref, comm_ref, rs_send, rs_recv, ag_send, ag_recv):
        my = lax.axis_index("i")
        left = lax.rem(my + N_DEV - 1, N_DEV)
        right = lax.rem(my + 1, N_DEV)

        barrier_sem = pltpu.get_barrier_semaphore()
        for nbr in (left, right):
            pl.semaphore_signal(
                barrier_sem, inc=1,
                device_id=(nbr,), device_id_type=pl.DeviceIdType.MESH,
            )
        pl.semaphore_wait(barrier_sem, 2)

        out_ref[:, :] = jnp.dot(
            x_ref[:, :], w_ref[:, :], preferred_element_type=jnp.float32
        )

        for s in range(N_DEV - 1):
            send_idx = lax.rem(my - s + N_DEV, N_DEV)
            rdma = pltpu.make_async_remote_copy(
                src_ref=out_ref.at[pl.ds(send_idx * ch, ch), :],
                dst_ref=comm_ref.at[s],
                send_sem=rs_send.at[s],
                recv_sem=rs_recv.at[s],
                device_id=(right,),
                device_id_type=pl.DeviceIdType.MESH,
            )
            rdma.start()
            rdma.wait()
            recv_idx = lax.rem(my - s - 1 + N_DEV, N_DEV)
            out_ref[pl.ds(recv_idx * ch, ch), :] += comm_ref[s]

        for s in range(N_DEV - 1):
            idx = lax.rem(my + 1 - s + 2 * N_DEV, N_DEV)
            rdma = pltpu.make_async_remote_copy(
                src_ref=out_ref.at[pl.ds(idx * ch, ch), :],
                dst_ref=out_ref.at[pl.ds(idx * ch, ch), :],
                send_sem=ag_send.at[s],
                recv_sem=ag_recv.at[s],
                device_id=(right,),
                device_id_type=pl.DeviceIdType.MESH,
            )
            rdma.start()
            rdma.wait()

    return pl.pallas_call(
        body,
        out_shape=jax.ShapeDtypeStruct((m, n), jnp.float32),
        in_specs=[
            pl.BlockSpec(memory_space=pltpu.VMEM),
            pl.BlockSpec(memory_space=pltpu.VMEM),
        ],
        out_specs=pl.BlockSpec(memory_space=pltpu.VMEM),
        scratch_shapes=[
            pltpu.VMEM((N_DEV - 1, m // N_DEV, n), jnp.float32),
            pltpu.SemaphoreType.DMA((N_DEV - 1,)),
            pltpu.SemaphoreType.DMA((N_DEV - 1,)),
            pltpu.SemaphoreType.DMA((N_DEV - 1,)),
            pltpu.SemaphoreType.DMA((N_DEV - 1,)),
        ],
        compiler_params=pltpu.CompilerParams(collective_id=0),
    )(x, w_mat)


# device time: 124515 ns/iter; 3.2752x vs baseline; 3.2752x over previous
import jax
import jax.numpy as jnp
from jax import lax
from jax.experimental import pallas as pl
from jax.experimental.pallas import tpu as pltpu

N_DEV = 16
NQ = 4
NH = N_DEV - 1


def kernel(x, w_mat):
    m, k = x.shape
    _, n = w_mat.shape
    ch = m // N_DEV
    half = n // 2
    qw = half // NQ

    def body(x_ref, w_ref, out_ref,
             rs_recv_R, rs_recv_L, rs_send_R, rs_send_L,
             ag_comm_R, ag_comm_L, ag0_R, ag0_L,
             sem_rs_s_R, sem_rs_r_R, sem_rs_s_L, sem_rs_r_L,
             sem_ag_s_R, sem_ag_r_R, sem_ag_s_L, sem_ag_r_L):
        my = lax.axis_index("i")
        left = lax.rem(my + N_DEV - 1, N_DEV)
        right = lax.rem(my + 1, N_DEV)

        barrier_sem = pltpu.get_barrier_semaphore()
        for nbr in (left, right):
            pl.semaphore_signal(
                barrier_sem, inc=1,
                device_id=(nbr,), device_id_type=pl.DeviceIdType.MESH,
            )
        pl.semaphore_wait(barrier_sem, 2)

        out_ref[:, :] = jnp.dot(
            x_ref[:, :], w_ref[:, :], preferred_element_type=jnp.float32
        )

        def rows(idx):
            return pl.ds(lax.rem(idx + 4 * N_DEV, N_DEV) * ch, ch)

        def cols(d, q):
            return pl.ds(d * half + q * qw, qw)

        def bufcols(q):
            return pl.ds(q * qw, qw)

        tgt = (right, left)
        rs_send = (rs_send_R, rs_send_L)
        rs_recv = (rs_recv_R, rs_recv_L)
        ag_comm = (ag_comm_R, ag_comm_L)
        ag0 = (ag0_R, ag0_L)
        sem_rs_s = (sem_rs_s_R, sem_rs_s_L)
        sem_rs_r = (sem_rs_r_R, sem_rs_r_L)
        sem_ag_s = (sem_ag_s_R, sem_ag_s_L)
        sem_ag_r = (sem_ag_r_R, sem_ag_r_L)

        def rs_desc(d, s, q):
            return pltpu.make_async_remote_copy(
                src_ref=rs_send[d].at[s, :, bufcols(q)],
                dst_ref=rs_recv[d].at[s, :, bufcols(q)],
                send_sem=sem_rs_s[d].at[s, q],
                recv_sem=sem_rs_r[d].at[s, q],
                device_id=(tgt[d],),
                device_id_type=pl.DeviceIdType.MESH,
            )

        def ag_desc(d, s, q):
            src = ag0[d].at[:, bufcols(q)] if s == 0 else (
                ag_comm[d].at[s - 1, :, bufcols(q)])
            return pltpu.make_async_remote_copy(
                src_ref=src,
                dst_ref=ag_comm[d].at[s, :, bufcols(q)],
                send_sem=sem_ag_s[d].at[s, q],
                recv_sem=sem_ag_r[d].at[s, q],
                device_id=(tgt[d],),
                device_id_type=pl.DeviceIdType.MESH,
            )

        rs = {}
        ag = {}

        for q in range(NQ):
            for d in (0, 1):
                rs_send[d][0, :, bufcols(q)] = (
                    out_ref[rows(my), cols(d, q)].astype(jnp.bfloat16))
                desc = rs_desc(d, 0, q)
                desc.start()
                rs[(d, 0, q)] = desc

        for s in range(1, NH):
            for q in range(NQ):
                for d in (0, 1):
                    ridx = my - s if d == 0 else my + s
                    rs[(d, s - 1, q)].wait_recv()
                    acc = (out_ref[rows(ridx), cols(d, q)]
                           + rs_recv[d][s - 1, :, bufcols(q)].astype(jnp.float32))
                    out_ref[rows(ridx), cols(d, q)] = acc
                    rs_send[d][s, :, bufcols(q)] = acc.astype(jnp.bfloat16)
                    desc = rs_desc(d, s, q)
                    desc.start()
                    rs[(d, s, q)] = desc

        for q in range(NQ):
            for d in (0, 1):
                ridx = my - NH if d == 0 else my + NH
                rs[(d, NH - 1, q)].wait_recv()
                acc = (out_ref[rows(ridx), cols(d, q)]
                       + rs_recv[d][NH - 1, :, bufcols(q)].astype(jnp.float32))
                out_ref[rows(ridx), cols(d, q)] = acc
                ag0[d][:, bufcols(q)] = acc.astype(jnp.bfloat16)
                desc = ag_desc(d, 0, q)
                desc.start()
                ag[(d, 0, q)] = desc

        for s in range(1, NH):
            for q in range(NQ):
                for d in (0, 1):
                    ridx = my - (s - 1) if d == 0 else my + (s - 1)
                    ag[(d, s - 1, q)].wait_recv()
                    desc = ag_desc(d, s, q)
                    desc.start()
                    ag[(d, s, q)] = desc
                    out_ref[rows(ridx), cols(d, q)] = (
                        ag_comm[d][s - 1, :, bufcols(q)].astype(jnp.float32))

        for q in range(NQ):
            for d in (0, 1):
                ridx = my - (NH - 1) if d == 0 else my + (NH - 1)
                ag[(d, NH - 1, q)].wait_recv()
                out_ref[rows(ridx), cols(d, q)] = (
                    ag_comm[d][NH - 1, :, bufcols(q)].astype(jnp.float32))

        for desc in list(rs.values()) + list(ag.values()):
            desc.wait_send()

    return pl.pallas_call(
        body,
        out_shape=jax.ShapeDtypeStruct((m, n), jnp.float32),
        in_specs=[
            pl.BlockSpec(memory_space=pltpu.VMEM),
            pl.BlockSpec(memory_space=pltpu.VMEM),
        ],
        out_specs=pl.BlockSpec(memory_space=pltpu.VMEM),
        scratch_shapes=[
            pltpu.VMEM((NH, m // N_DEV, n // 2), jnp.bfloat16),
            pltpu.VMEM((NH, m // N_DEV, n // 2), jnp.bfloat16),
            pltpu.VMEM((NH, m // N_DEV, n // 2), jnp.bfloat16),
            pltpu.VMEM((NH, m // N_DEV, n // 2), jnp.bfloat16),
            pltpu.VMEM((NH, m // N_DEV, n // 2), jnp.bfloat16),
            pltpu.VMEM((NH, m // N_DEV, n // 2), jnp.bfloat16),
            pltpu.VMEM((m // N_DEV, n // 2), jnp.bfloat16),
            pltpu.VMEM((m // N_DEV, n // 2), jnp.bfloat16),
            pltpu.SemaphoreType.DMA((NH, NQ)),
            pltpu.SemaphoreType.DMA((NH, NQ)),
            pltpu.SemaphoreType.DMA((NH, NQ)),
            pltpu.SemaphoreType.DMA((NH, NQ)),
            pltpu.SemaphoreType.DMA((NH, NQ)),
            pltpu.SemaphoreType.DMA((NH, NQ)),
            pltpu.SemaphoreType.DMA((NH, NQ)),
            pltpu.SemaphoreType.DMA((NH, NQ)),
        ],
        compiler_params=pltpu.CompilerParams(collective_id=0),
    )(x, w_mat)


# device time: 123956 ns/iter; 3.2899x vs baseline; 1.0045x over previous
import jax
import jax.numpy as jnp
from jax import lax
from jax.experimental import pallas as pl
from jax.experimental.pallas import tpu as pltpu

N_DEV = 16
NQ = 4
NH = N_DEV - 1


def kernel(x, w_mat):
    m, k = x.shape
    _, n = w_mat.shape
    ch = m // N_DEV
    half = n // 2
    qw = half // NQ

    def body(x_ref, w_ref, out_ref,
             rs_recv_R, rs_recv_L, rs_send_R, rs_send_L,
             ag_comm_R, ag_comm_L, ag0_R, ag0_L,
             sem_rs_s_R, sem_rs_r_R, sem_rs_s_L, sem_rs_r_L,
             sem_ag_s_R, sem_ag_r_R, sem_ag_s_L, sem_ag_r_L):
        my = lax.axis_index("i")
        left = lax.rem(my + N_DEV - 1, N_DEV)
        right = lax.rem(my + 1, N_DEV)

        barrier_sem = pltpu.get_barrier_semaphore()
        for nbr in (left, right):
            pl.semaphore_signal(
                barrier_sem, inc=1,
                device_id=(nbr,), device_id_type=pl.DeviceIdType.MESH,
            )
        pl.semaphore_wait(barrier_sem, 2)

        out_ref[:, :] = jnp.dot(
            x_ref[:, :], w_ref[:, :], preferred_element_type=jnp.float32
        )

        def rows(idx):
            return pl.ds(lax.rem(idx + 4 * N_DEV, N_DEV) * ch, ch)

        def cols(d, q):
            return pl.ds(d * half + q * qw, qw)

        def bufcols(q):
            return pl.ds(q * qw, qw)

        tgt = (right, left)
        rs_send = (rs_send_R, rs_send_L)
        rs_recv = (rs_recv_R, rs_recv_L)
        ag_comm = (ag_comm_R, ag_comm_L)
        ag0 = (ag0_R, ag0_L)
        sem_rs_s = (sem_rs_s_R, sem_rs_s_L)
        sem_rs_r = (sem_rs_r_R, sem_rs_r_L)
        sem_ag_s = (sem_ag_s_R, sem_ag_s_L)
        sem_ag_r = (sem_ag_r_R, sem_ag_r_L)

        def rs_desc(d, s, q):
            return pltpu.make_async_remote_copy(
                src_ref=rs_send[d].at[s, :, bufcols(q)],
                dst_ref=rs_recv[d].at[s, :, bufcols(q)],
                send_sem=sem_rs_s[d].at[s, q],
                recv_sem=sem_rs_r[d].at[s, q],
                device_id=(tgt[d],),
                device_id_type=pl.DeviceIdType.MESH,
            )

        def ag_desc(d, s, q):
            src = ag0[d].at[:, bufcols(q)] if s == 0 else (
                ag_comm[d].at[s - 1, :, bufcols(q)])
            return pltpu.make_async_remote_copy(
                src_ref=src,
                dst_ref=ag_comm[d].at[s, :, bufcols(q)],
                send_sem=sem_ag_s[d].at[s, q],
                recv_sem=sem_ag_r[d].at[s, q],
                device_id=(tgt[d],),
                device_id_type=pl.DeviceIdType.MESH,
            )

        rs = {}
        ag = {}

        for q in range(NQ):
            for d in (0, 1):
                rs_send[d][0, :, bufcols(q)] = (
                    out_ref[rows(my), cols(d, q)].astype(jnp.bfloat16))
                desc = rs_desc(d, 0, q)
                desc.start()
                rs[(d, 0, q)] = desc

        for s in range(1, NH):
            for q in range(NQ):
                for d in (0, 1):
                    ridx = my - s if d == 0 else my + s
                    rs[(d, s - 1, q)].wait_recv()
                    rs_send[d][s, :, bufcols(q)] = (
                        rs_recv[d][s - 1, :, bufcols(q)].astype(jnp.float32)
                        + out_ref[rows(ridx), cols(d, q)]
                    ).astype(jnp.bfloat16)
                    desc = rs_desc(d, s, q)
                    desc.start()
                    rs[(d, s, q)] = desc

        for q in range(NQ):
            for d in (0, 1):
                ridx = my - NH if d == 0 else my + NH
                rs[(d, NH - 1, q)].wait_recv()
                acc = (out_ref[rows(ridx), cols(d, q)]
                       + rs_recv[d][NH - 1, :, bufcols(q)].astype(jnp.float32))
                out_ref[rows(ridx), cols(d, q)] = acc
                ag0[d][:, bufcols(q)] = acc.astype(jnp.bfloat16)
                desc = ag_desc(d, 0, q)
                desc.start()
                ag[(d, 0, q)] = desc

        for s in range(1, NH):
            for q in range(NQ):
                for d in (0, 1):
                    ridx = my - (s - 1) if d == 0 else my + (s - 1)
                    ag[(d, s - 1, q)].wait_recv()
                    desc = ag_desc(d, s, q)
                    desc.start()
                    ag[(d, s, q)] = desc
                    out_ref[rows(ridx), cols(d, q)] = (
                        ag_comm[d][s - 1, :, bufcols(q)].astype(jnp.float32))

        for q in range(NQ):
            for d in (0, 1):
                ridx = my - (NH - 1) if d == 0 else my + (NH - 1)
                ag[(d, NH - 1, q)].wait_recv()
                out_ref[rows(ridx), cols(d, q)] = (
                    ag_comm[d][NH - 1, :, bufcols(q)].astype(jnp.float32))

        for desc in list(rs.values()) + list(ag.values()):
            desc.wait_send()

    return pl.pallas_call(
        body,
        out_shape=jax.ShapeDtypeStruct((m, n), jnp.float32),
        in_specs=[
            pl.BlockSpec(memory_space=pltpu.VMEM),
            pl.BlockSpec(memory_space=pltpu.VMEM),
        ],
        out_specs=pl.BlockSpec(memory_space=pltpu.VMEM),
        scratch_shapes=[
            pltpu.VMEM((NH, m // N_DEV, n // 2), jnp.bfloat16),
            pltpu.VMEM((NH, m // N_DEV, n // 2), jnp.bfloat16),
            pltpu.VMEM((NH, m // N_DEV, n // 2), jnp.bfloat16),
            pltpu.VMEM((NH, m // N_DEV, n // 2), jnp.bfloat16),
            pltpu.VMEM((NH, m // N_DEV, n // 2), jnp.bfloat16),
            pltpu.VMEM((NH, m // N_DEV, n // 2), jnp.bfloat16),
            pltpu.VMEM((m // N_DEV, n // 2), jnp.bfloat16),
            pltpu.VMEM((m // N_DEV, n // 2), jnp.bfloat16),
            pltpu.SemaphoreType.DMA((NH, NQ)),
            pltpu.SemaphoreType.DMA((NH, NQ)),
            pltpu.SemaphoreType.DMA((NH, NQ)),
            pltpu.SemaphoreType.DMA((NH, NQ)),
            pltpu.SemaphoreType.DMA((NH, NQ)),
            pltpu.SemaphoreType.DMA((NH, NQ)),
            pltpu.SemaphoreType.DMA((NH, NQ)),
            pltpu.SemaphoreType.DMA((NH, NQ)),
        ],
        compiler_params=pltpu.CompilerParams(collective_id=0),
    )(x, w_mat)


# device time: 106445 ns/iter; 3.8311x vs baseline; 1.1645x over previous
import jax
import jax.numpy as jnp
from jax import lax
from jax.experimental import pallas as pl
from jax.experimental.pallas import tpu as pltpu

N_DEV = 16
NQ = 4
NH = N_DEV - 1

_PERM = (0, 4, 8, 12, 15, 11, 7, 3, 2, 6, 10, 14, 13, 9, 5, 1)
_PERM_INV = tuple(_PERM.index(i) for i in range(N_DEV))


def kernel(x, w_mat):
    m, k = x.shape
    _, n = w_mat.shape
    ch = m // N_DEV
    half = n // 2
    qw = half // NQ

    my = lax.axis_index("i")
    pos = jnp.take(jnp.array(_PERM_INV, jnp.int32), my)
    nxt = jnp.take(jnp.array(_PERM, jnp.int32), (pos + 1) % N_DEV)
    prv = jnp.take(jnp.array(_PERM, jnp.int32), (pos + N_DEV - 1) % N_DEV)
    meta = jnp.stack([pos, nxt, prv]).astype(jnp.int32)

    def body(meta_ref, x_ref, w_ref, out_ref,
             part16,
             rs_recv_R, rs_recv_L, rs_send_R, rs_send_L,
             ag_comm_R, ag_comm_L, ag0_R, ag0_L,
             sem_rs_s_R, sem_rs_r_R, sem_rs_s_L, sem_rs_r_L,
             sem_ag_s_R, sem_ag_r_R, sem_ag_s_L, sem_ag_r_L):
        pos = meta_ref[0]
        right = meta_ref[1]
        left = meta_ref[2]

        barrier_sem = pltpu.get_barrier_semaphore()
        for nbr in (left, right):
            pl.semaphore_signal(
                barrier_sem, inc=1,
                device_id=(nbr,), device_id_type=pl.DeviceIdType.MESH,
            )
        pl.semaphore_wait(barrier_sem, 2)

        out_ref[:, :] = jnp.dot(
            x_ref[:, :], w_ref[:, :], preferred_element_type=jnp.float32
        )
        part16[:, :] = out_ref[:, :].astype(jnp.bfloat16)

        def rows(idx):
            return pl.ds(lax.rem(idx + 4 * N_DEV, N_DEV) * ch, ch)

        def cols(d, q):
            return pl.ds(d * half + q * qw, qw)

        def bufcols(q):
            return pl.ds(q * qw, qw)

        tgt = (right, left)
        rs_send = (rs_send_R, rs_send_L)
        rs_recv = (rs_recv_R, rs_recv_L)
        ag_comm = (ag_comm_R, ag_comm_L)
        ag0 = (ag0_R, ag0_L)
        sem_rs_s = (sem_rs_s_R, sem_rs_s_L)
        sem_rs_r = (sem_rs_r_R, sem_rs_r_L)
        sem_ag_s = (sem_ag_s_R, sem_ag_s_L)
        sem_ag_r = (sem_ag_r_R, sem_ag_r_L)

        def rs_desc(d, s, q):
            return pltpu.make_async_remote_copy(
                src_ref=rs_send[d].at[s, :, bufcols(q)],
                dst_ref=rs_recv[d].at[s, :, bufcols(q)],
                send_sem=sem_rs_s[d].at[s, q],
                recv_sem=sem_rs_r[d].at[s, q],
                device_id=(tgt[d],),
                device_id_type=pl.DeviceIdType.MESH,
            )

        def ag_desc(d, s, q):
            src = ag0[d].at[:, bufcols(q)] if s == 0 else (
                ag_comm[d].at[s - 1, :, bufcols(q)])
            return pltpu.make_async_remote_copy(
                src_ref=src,
                dst_ref=ag_comm[d].at[s, :, bufcols(q)],
                send_sem=sem_ag_s[d].at[s, q],
                recv_sem=sem_ag_r[d].at[s, q],
                device_id=(tgt[d],),
                device_id_type=pl.DeviceIdType.MESH,
            )

        rs = {}
        ag = {}

        for q in range(NQ):
            for d in (0, 1):
                rs_send[d][0, :, bufcols(q)] = part16[rows(pos), cols(d, q)]
                desc = rs_desc(d, 0, q)
                desc.start()
                rs[(d, 0, q)] = desc

        for s in range(1, NH):
            for q in range(NQ):
                for d in (0, 1):
                    ridx = pos - s if d == 0 else pos + s
                    rs[(d, s - 1, q)].wait_recv()
                    rs_send[d][s, :, bufcols(q)] = (
                        rs_recv[d][s - 1, :, bufcols(q)]
                        + part16[rows(ridx), cols(d, q)])
                    desc = rs_desc(d, s, q)
                    desc.start()
                    rs[(d, s, q)] = desc

        for q in range(NQ):
            for d in (0, 1):
                ridx = pos - NH if d == 0 else pos + NH
                rs[(d, NH - 1, q)].wait_recv()
                acc = (out_ref[rows(ridx), cols(d, q)]
                       + rs_recv[d][NH - 1, :, bufcols(q)].astype(jnp.float32))
                out_ref[rows(ridx), cols(d, q)] = acc
                ag0[d][:, bufcols(q)] = acc.astype(jnp.bfloat16)
                desc = ag_desc(d, 0, q)
                desc.start()
                ag[(d, 0, q)] = desc

        for s in range(1, NH):
            for q in range(NQ):
                for d in (0, 1):
                    ridx = pos - (s - 1) if d == 0 else pos + (s - 1)
                    ag[(d, s - 1, q)].wait_recv()
                    desc = ag_desc(d, s, q)
                    desc.start()
                    ag[(d, s, q)] = desc
                    out_ref[rows(ridx), cols(d, q)] = (
                        ag_comm[d][s - 1, :, bufcols(q)].astype(jnp.float32))

        for q in range(NQ):
            for d in (0, 1):
                ridx = pos - (NH - 1) if d == 0 else pos + (NH - 1)
                ag[(d, NH - 1, q)].wait_recv()
                out_ref[rows(ridx), cols(d, q)] = (
                    ag_comm[d][NH - 1, :, bufcols(q)].astype(jnp.float32))

        for desc in list(rs.values()) + list(ag.values()):
            desc.wait_send()

    return pl.pallas_call(
        body,
        out_shape=jax.ShapeDtypeStruct((m, n), jnp.float32),
        in_specs=[
            pl.BlockSpec(memory_space=pltpu.SMEM),
            pl.BlockSpec(memory_space=pltpu.VMEM),
            pl.BlockSpec(memory_space=pltpu.VMEM),
        ],
        out_specs=pl.BlockSpec(memory_space=pltpu.VMEM),
        scratch_shapes=[
            pltpu.VMEM((m, n), jnp.bfloat16),
            pltpu.VMEM((NH, m // N_DEV, n // 2), jnp.bfloat16),
            pltpu.VMEM((NH, m // N_DEV, n // 2), jnp.bfloat16),
            pltpu.VMEM((NH, m // N_DEV, n // 2), jnp.bfloat16),
            pltpu.VMEM((NH, m // N_DEV, n // 2), jnp.bfloat16),
            pltpu.VMEM((NH, m // N_DEV, n // 2), jnp.bfloat16),
            pltpu.VMEM((NH, m // N_DEV, n // 2), jnp.bfloat16),
            pltpu.VMEM((m // N_DEV, n // 2), jnp.bfloat16),
            pltpu.VMEM((m // N_DEV, n // 2), jnp.bfloat16),
            pltpu.SemaphoreType.DMA((NH, NQ)),
            pltpu.SemaphoreType.DMA((NH, NQ)),
            pltpu.SemaphoreType.DMA((NH, NQ)),
            pltpu.SemaphoreType.DMA((NH, NQ)),
            pltpu.SemaphoreType.DMA((NH, NQ)),
            pltpu.SemaphoreType.DMA((NH, NQ)),
            pltpu.SemaphoreType.DMA((NH, NQ)),
            pltpu.SemaphoreType.DMA((NH, NQ)),
        ],
        compiler_params=pltpu.CompilerParams(collective_id=0),
    )(meta, x, w_mat)
